# baseline (speedup 1.0000x reference)
import functools

import jax
import jax.numpy as jnp
from jax import lax
from jax.experimental import pallas as pl
from jax.experimental.pallas import tpu as pltpu
from jax.experimental.pallas import tpu_sc as plsc

_B = 16384
_D = 32
_NSRC = 10000
_L = 16

_FR = 4
_NF = _D // _FR
_NH = 2
_BQ = _B // _NH


@functools.partial(
    pl.kernel,
    out_type=(
        jax.ShapeDtypeStruct((_D, _B), jnp.int32),
        jax.ShapeDtypeStruct((_D, _B), jnp.float32),
    ),
    mesh=plsc.VectorSubcoreMesh(core_axis_name="c", subcore_axis_name="s"),
    scratch_types=[
        pltpu.VMEM((_BQ,), jnp.int32),
    ],
    compiler_params=pltpu.CompilerParams(use_tc_tiling_on_sc=True, disable_bounds_checks=True, disable_semaphore_checks=True, skip_device_barrier=True,
                                         needs_layout_passes=False),
)
def _gather2(idx_hbm, tgtT_hbm, wT_hbm, out_t, out_w, idx_v):
    w = lax.axis_index("s") * 2 + lax.axis_index("c")
    t = w // 16
    f = (w // _NH) % _NF
    h = w % _NH

    def run(table_hbm, out_hbm, dtype):
        def scoped(slab_v, res_v, sem):
            c_idx = pltpu.async_copy(
                idx_hbm.at[pl.ds(h * _BQ, _BQ)], idx_v, sem)
            c_slab = pltpu.async_copy(table_hbm.at[pl.ds(_FR * f, _FR)],
                                      slab_v, sem)
            c_idx.wait()
            c_slab.wait()

            ck = _BQ // 2
            outs = []
            for c in range(2):
                @plsc.parallel_loop(c * (ck // _L), (c + 1) * (ck // _L),
                                    unroll=4)
                def _gather_loop(i):
                    iv = idx_v[pl.ds(i * _L, _L)]
                    for j in range(_FR):
                        jv = jnp.full((_L,), j, jnp.int32)
                        res_v[j, pl.ds(i * _L, _L)] = plsc.load_gather(
                            slab_v, [jv, iv])

                outs.append(pltpu.async_copy(
                    res_v.at[:, pl.ds(c * ck, ck)],
                    out_hbm.at[pl.ds(_FR * f, _FR),
                               pl.ds(h * _BQ + c * ck, ck)],
                    sem))
            for o in outs:
                o.wait()

        pl.run_scoped(scoped,
                      pltpu.VMEM((_FR, _NSRC), dtype),
                      pltpu.VMEM((_FR, _BQ), dtype),
                      pltpu.SemaphoreType.DMA)

    @pl.when(t == 0)
    def _table_t():
        run(tgtT_hbm, out_t, jnp.int32)

    @pl.when(t == 1)
    def _table_w():
        run(wT_hbm, out_w, jnp.float32)


def kernel(src_neurons, cached_targets, weights):
    outTt, outTw = _gather2(src_neurons.astype(jnp.int32),
                            cached_targets.T, weights.T)
    return outTt.T, outTw.T


# speedup vs baseline: 1.0045x; 1.0045x over previous
import functools

import jax
import jax.numpy as jnp
from jax import lax
from jax.experimental import pallas as pl
from jax.experimental.pallas import tpu as pltpu
from jax.experimental.pallas import tpu_sc as plsc

_B = 16384
_D = 32
_NSRC = 10000
_L = 16

_FR = 4
_NF = _D // _FR
_NH = 2
_BQ = _B // _NH


@functools.partial(
    pl.kernel,
    out_type=(
        jax.ShapeDtypeStruct((_D, _B), jnp.int32),
        jax.ShapeDtypeStruct((_D, _B), jnp.float32),
    ),
    mesh=plsc.VectorSubcoreMesh(core_axis_name="c", subcore_axis_name="s"),
    scratch_types=[
        pltpu.VMEM((_BQ,), jnp.int32),
    ],
    compiler_params=pltpu.CompilerParams(use_tc_tiling_on_sc=True,
                                         needs_layout_passes=False),
)
def _gather2(idx_hbm, tgtT_hbm, wT_hbm, out_t, out_w, idx_v):
    w = lax.axis_index("s") * 2 + lax.axis_index("c")
    t = w // 16
    f = (w // _NH) % _NF
    h = w % _NH

    def run(table_hbm, out_hbm, dtype):
        def scoped(slab_v, res_v, sem):
            c_idx = pltpu.async_copy(
                idx_hbm.at[pl.ds(h * _BQ, _BQ)], idx_v, sem)
            c_slab = pltpu.async_copy(table_hbm.at[pl.ds(_FR * f, _FR)],
                                      slab_v, sem)
            c_idx.wait()
            c_slab.wait()

            ck = _BQ // 2
            outs = []
            for c in range(2):
                @plsc.parallel_loop(c * (ck // _L), (c + 1) * (ck // _L),
                                    unroll=8)
                def _gather_loop(i):
                    iv = idx_v[pl.ds(i * _L, _L)]
                    for j in range(_FR):
                        jv = jnp.full((_L,), j, jnp.int32)
                        res_v[j, pl.ds(i * _L, _L)] = plsc.load_gather(
                            slab_v, [jv, iv])

                outs.append(pltpu.async_copy(
                    res_v.at[:, pl.ds(c * ck, ck)],
                    out_hbm.at[pl.ds(_FR * f, _FR),
                               pl.ds(h * _BQ + c * ck, ck)],
                    sem))
            for o in outs:
                o.wait()

        pl.run_scoped(scoped,
                      pltpu.VMEM((_FR, _NSRC), dtype),
                      pltpu.VMEM((_FR, _BQ), dtype),
                      pltpu.SemaphoreType.DMA)

    @pl.when(t == 0)
    def _table_t():
        run(tgtT_hbm, out_t, jnp.int32)

    @pl.when(t == 1)
    def _table_w():
        run(wT_hbm, out_w, jnp.float32)


def kernel(src_neurons, cached_targets, weights):
    outTt, outTw = _gather2(src_neurons.astype(jnp.int32),
                            cached_targets.T, weights.T)
    return outTt.T, outTw.T
